# baseline (speedup 1.0000x reference)
import functools

import jax
import jax.numpy as jnp
from jax import lax
from jax.experimental import pallas as pl
from jax.experimental.pallas import tpu as pltpu
from jax.experimental.pallas import tpu_sc as plsc

N = 1048576
D = 32
D_OBS = 8
D_NA = 24
L = 16
NW = 32
TILE_TC = 2048
N_SC = 524288
N_TC = N - N_SC
ROWS_PER_W = N_SC // NW
CH = 2048
NCHUNK = ROWS_PER_W // CH
GRP = CH // L
G = 4
ITERS = GRP // G

_LOG2_POLY = (
    2.1309029207827734e-08, 1.4426926001264684, -0.7212782647852086,
    0.48004523877905203, -0.35498277313144655, 0.2654635523085577,
    -0.17940063279201465, 0.09493297200142276, -0.03281798496137954,
    0.005345286470990158,
)
_LN2 = 0.6931471805599453


def _fast_log2(p):
    bits = plsc.bitcast(p, jnp.int32)
    e = lax.shift_right_logical(bits, 23) - 127
    m = plsc.bitcast((bits & 0x007FFFFF) | 0x3F800000, jnp.float32)
    r = m - 1.0
    poly = jnp.full((L,), _LOG2_POLY[-1], jnp.float32)
    for c in _LOG2_POLY[-2::-1]:
        poly = poly * r + jnp.float32(c)
    return e.astype(jnp.float32) + poly


def _body(xt_hbm, ctab_hbm, out_hbm, xb0, xb1, cb, ob0, ob1, s0, s1, so0, so1):
    wid = lax.axis_index("s") * 2 + lax.axis_index("c")
    pltpu.sync_copy(ctab_hbm, cb)
    base0 = N_TC + wid * ROWS_PER_W

    def start_fetch(c, xb, sem):
        src = xt_hbm.at[pl.ds(0, D_OBS), pl.ds(base0 + c * CH, CH)]
        pltpu.async_copy(src, xb, sem)

    def wait_fetch(xb, sem):
        src = xt_hbm.at[pl.ds(0, D_OBS), pl.ds(0, CH)]
        pltpu.make_async_copy(src, xb, sem).wait()

    def compute_chunk(xb, chunk, ob, osem):
        @pl.when(chunk >= 2)
        def _wait_prev_out():
            pltpu.make_async_copy(ob, out_hbm.at[pl.ds(0, CH)], osem).wait()

        def grp_body(it, carry):
            row0 = it * (L * G)
            xk = [
                [xb[k, pl.ds(row0 + i * L, L)] for k in range(D_OBS)]
                for i in range(G)
            ]
            xp = [
                [plsc.pack(xk[2 * p][k], xk[2 * p + 1][k],
                           format=plsc.PackFormat.INTERLEAVED)
                 for k in range(D_OBS)]
                for p in range(G // 2)
            ]
            one2 = jnp.full((2 * L,), 1.0, jnp.bfloat16)
            prod = [one2 for _ in range(G // 2)]
            ssum = [jnp.zeros((2 * L,), jnp.bfloat16) for _ in range(G // 2)]
            for j in range(D_NA):
                cv = [plsc.bitcast(cb[pl.ds((j * (D_OBS + 1) + k) * L, L)],
                                   jnp.bfloat16)
                      for k in range(D_OBS + 1)]
                for p in range(G // 2):
                    z2 = cv[D_OBS]
                    for k in range(D_OBS):
                        z2 = z2 + xp[p][k] * cv[k]
                    t2 = jnp.exp(-jnp.abs(z2))
                    prod[p] = prod[p] * (one2 + t2)
                    ssum[p] = ssum[p] + jnp.maximum(z2, jnp.bfloat16(0.0))
            for p in range(G // 2):
                pu = plsc.unpack(prod[p], format=plsc.PackFormat.INTERLEAVED)
                su = plsc.unpack(ssum[p], format=plsc.PackFormat.INTERLEAVED)
                for h in range(2):
                    res = -(su[h].astype(jnp.float32)
                            + _fast_log2(pu[h].astype(jnp.float32))
                            * jnp.float32(_LN2))
                    ob[pl.ds(row0 + (2 * p + h) * L, L)] = res
            return carry

        lax.fori_loop(0, ITERS, grp_body, 0, unroll=False)
        pltpu.async_copy(
            ob,
            out_hbm.at[pl.ds(base0 - N_TC + chunk * CH, CH)],
            osem,
        )

    start_fetch(0, xb0, s0)

    def step(s, carry):
        c0 = 2 * s
        wait_fetch(xb0, s0)
        start_fetch(c0 + 1, xb1, s1)
        compute_chunk(xb0, c0, ob0, so0)
        wait_fetch(xb1, s1)
        start_fetch(jnp.minimum(c0 + 2, NCHUNK - 1), xb0, s0)
        compute_chunk(xb1, c0 + 1, ob1, so1)
        return carry

    lax.fori_loop(0, NCHUNK // 2, step, 0, unroll=False)
    wait_fetch(xb0, s0)
    pltpu.make_async_copy(ob0, out_hbm.at[pl.ds(0, CH)], so0).wait()
    pltpu.make_async_copy(ob1, out_hbm.at[pl.ds(0, CH)], so1).wait()


_mesh = plsc.VectorSubcoreMesh(core_axis_name="c", subcore_axis_name="s")

_sc_call = functools.partial(
    pl.kernel,
    out_type=jax.ShapeDtypeStruct((N_SC,), jnp.float32),
    mesh=_mesh,
    compiler_params=pltpu.CompilerParams(needs_layout_passes=False),
    scratch_types=[
        pltpu.VMEM((D_OBS, CH), jnp.float32),
        pltpu.VMEM((D_OBS, CH), jnp.float32),
        pltpu.VMEM((D_NA * (D_OBS + 1) * L,), jnp.int32),
        pltpu.VMEM((CH,), jnp.float32),
        pltpu.VMEM((CH,), jnp.float32),
        pltpu.SemaphoreType.DMA,
        pltpu.SemaphoreType.DMA,
        pltpu.SemaphoreType.DMA,
        pltpu.SemaphoreType.DMA,
    ],
)(_body)


def _tc_body(xt_ref, c_ref, b_ref, o_ref):
    x = xt_ref[...]
    z = lax.dot_general(
        c_ref[...], x, dimension_numbers=(((0,), (0,)), ((), ())),
        preferred_element_type=jnp.float32,
    )
    z = z + b_ref[...]
    t = jnp.exp(-jnp.abs(z))
    s = jnp.sum(jnp.maximum(z, 0.0), axis=0)
    pr = 1.0 + t
    pr = pr[:12] * pr[12:]
    pr = pr[:6] * pr[6:]
    pr = pr[:3] * pr[3:]
    pr = pr[0] * pr[1] * pr[2]
    o_ref[...] = -(s + jnp.log(pr))


_tc_call = pl.pallas_call(
    _tc_body,
    grid=(N_TC // TILE_TC,),
    in_specs=[
        pl.BlockSpec((D_OBS, TILE_TC), lambda i: (0, i)),
        pl.BlockSpec((D_OBS, D_NA), lambda i: (0, 0)),
        pl.BlockSpec((D_NA, 1), lambda i: (0, 0)),
    ],
    out_specs=pl.BlockSpec((TILE_TC,), lambda i: (i,)),
    out_shape=jax.ShapeDtypeStruct((N_TC,), jnp.float32),
)


def kernel(X, M, idxs_params, idxs_miss, coeffs, intercepts):
    order = jnp.argsort(idxs_params)
    ctab = jnp.concatenate([coeffs[order], intercepts[None, :]], axis=0)
    bits = lax.bitcast_convert_type(ctab.T.astype(jnp.bfloat16), jnp.uint16)
    word = bits.astype(jnp.uint32) | (bits.astype(jnp.uint32) << 16)
    ctab = jnp.broadcast_to(
        lax.bitcast_convert_type(word, jnp.int32).reshape(
            D_NA * (D_OBS + 1), 1), (D_NA * (D_OBS + 1), L)
    ).reshape(D_NA * (D_OBS + 1) * L)
    xt = X.T
    out_sc = _sc_call(xt, ctab)
    out_tc = _tc_call(xt, coeffs[order].astype(jnp.float32),
                      intercepts.astype(jnp.float32)[:, None])
    return jnp.concatenate([out_tc, out_sc])


# speedup vs baseline: 1.4606x; 1.4606x over previous
import functools

import jax
import jax.numpy as jnp
from jax import lax
from jax.experimental import pallas as pl
from jax.experimental.pallas import tpu as pltpu
from jax.experimental.pallas import tpu_sc as plsc

N = 1048576
D = 32
D_OBS = 8
D_NA = 24
L = 16
NW = 32
TILE_TC = 2048
N_SC = 720896
N_TC = N - N_SC
ROWS_PER_W = N_SC // NW
CH = 1024
NCHUNK = ROWS_PER_W // CH
GRP = CH // L
G = 4
ITERS = GRP // G

_LOG2_POLY = (
    2.1309029207827734e-08, 1.4426926001264684, -0.7212782647852086,
    0.48004523877905203, -0.35498277313144655, 0.2654635523085577,
    -0.17940063279201465, 0.09493297200142276, -0.03281798496137954,
    0.005345286470990158,
)
_LN2 = 0.6931471805599453


def _fast_log2(p):
    bits = plsc.bitcast(p, jnp.int32)
    e = lax.shift_right_logical(bits, 23) - 127
    m = plsc.bitcast((bits & 0x007FFFFF) | 0x3F800000, jnp.float32)
    r = m - 1.0
    poly = jnp.full((L,), _LOG2_POLY[-1], jnp.float32)
    for c in _LOG2_POLY[-2::-1]:
        poly = poly * r + jnp.float32(c)
    return e.astype(jnp.float32) + poly


def _body(xt_hbm, ctab_hbm, out_hbm, xb0, xb1, cb, ob0, ob1, s0, s1, so0, so1):
    wid = lax.axis_index("s") * 2 + lax.axis_index("c")
    pltpu.sync_copy(ctab_hbm, cb)
    base0 = N_TC + wid * ROWS_PER_W

    def start_fetch(c, xb, sem):
        src = xt_hbm.at[pl.ds(0, D_OBS), pl.ds(base0 + c * CH, CH)]
        pltpu.async_copy(src, xb, sem)

    def wait_fetch(xb, sem):
        src = xt_hbm.at[pl.ds(0, D_OBS), pl.ds(0, CH)]
        pltpu.make_async_copy(src, xb, sem).wait()

    def compute_chunk(xb, chunk, ob, osem):
        @pl.when(chunk >= 2)
        def _wait_prev_out():
            pltpu.make_async_copy(ob, out_hbm.at[pl.ds(0, CH)], osem).wait()

        def grp_body(it, carry):
            row0 = it * (L * G)
            xk = [
                [xb[k, pl.ds(row0 + i * L, L)] for k in range(D_OBS)]
                for i in range(G)
            ]
            xp = [
                [plsc.pack(xk[2 * p][k], xk[2 * p + 1][k],
                           format=plsc.PackFormat.INTERLEAVED)
                 for k in range(D_OBS)]
                for p in range(G // 2)
            ]
            one2 = jnp.full((2 * L,), 1.0, jnp.bfloat16)
            prod = [one2 for _ in range(G // 2)]
            ssum = [jnp.zeros((2 * L,), jnp.bfloat16) for _ in range(G // 2)]
            for j in range(D_NA):
                cv = [plsc.bitcast(cb[pl.ds((j * (D_OBS + 1) + k) * L, L)],
                                   jnp.bfloat16)
                      for k in range(D_OBS + 1)]
                for p in range(G // 2):
                    z2 = cv[D_OBS]
                    for k in range(D_OBS):
                        z2 = z2 + xp[p][k] * cv[k]
                    t2 = jnp.exp(-jnp.abs(z2))
                    prod[p] = prod[p] * (one2 + t2)
                    ssum[p] = ssum[p] + jnp.maximum(z2, jnp.bfloat16(0.0))
            for p in range(G // 2):
                pu = plsc.unpack(prod[p], format=plsc.PackFormat.INTERLEAVED)
                su = plsc.unpack(ssum[p], format=plsc.PackFormat.INTERLEAVED)
                for h in range(2):
                    res = -(su[h].astype(jnp.float32)
                            + _fast_log2(pu[h].astype(jnp.float32))
                            * jnp.float32(_LN2))
                    ob[pl.ds(row0 + (2 * p + h) * L, L)] = res
            return carry

        lax.fori_loop(0, ITERS, grp_body, 0, unroll=False)
        pltpu.async_copy(
            ob,
            out_hbm.at[pl.ds(base0 - N_TC + chunk * CH, CH)],
            osem,
        )

    start_fetch(0, xb0, s0)

    def step(s, carry):
        c0 = 2 * s
        wait_fetch(xb0, s0)
        start_fetch(c0 + 1, xb1, s1)
        compute_chunk(xb0, c0, ob0, so0)
        wait_fetch(xb1, s1)
        start_fetch(jnp.minimum(c0 + 2, NCHUNK - 1), xb0, s0)
        compute_chunk(xb1, c0 + 1, ob1, so1)
        return carry

    lax.fori_loop(0, NCHUNK // 2, step, 0, unroll=False)
    wait_fetch(xb0, s0)
    pltpu.make_async_copy(ob0, out_hbm.at[pl.ds(0, CH)], so0).wait()
    pltpu.make_async_copy(ob1, out_hbm.at[pl.ds(0, CH)], so1).wait()


_mesh = plsc.VectorSubcoreMesh(core_axis_name="c", subcore_axis_name="s")

_sc_call = functools.partial(
    pl.kernel,
    out_type=jax.ShapeDtypeStruct((N_SC,), jnp.float32),
    mesh=_mesh,
    compiler_params=pltpu.CompilerParams(needs_layout_passes=False),
    scratch_types=[
        pltpu.VMEM((D_OBS, CH), jnp.float32),
        pltpu.VMEM((D_OBS, CH), jnp.float32),
        pltpu.VMEM((D_NA * (D_OBS + 1) * L,), jnp.int32),
        pltpu.VMEM((CH,), jnp.float32),
        pltpu.VMEM((CH,), jnp.float32),
        pltpu.SemaphoreType.DMA,
        pltpu.SemaphoreType.DMA,
        pltpu.SemaphoreType.DMA,
        pltpu.SemaphoreType.DMA,
    ],
)(_body)


def _tc_body(xt_ref, c_ref, b_ref, o_ref):
    x = xt_ref[...]
    z = lax.dot_general(
        c_ref[...], x, dimension_numbers=(((0,), (0,)), ((), ())),
        preferred_element_type=jnp.float32,
    )
    z = z + b_ref[...]
    s = jnp.sum(jnp.maximum(z, 0.0), axis=0)
    zb = z.astype(jnp.bfloat16)
    t = jnp.exp(-jnp.abs(zb))
    pr = jnp.bfloat16(1.0) + t
    pr = pr[:12] * pr[12:]
    pr = pr[:6] * pr[6:]
    pr = pr[:3] * pr[3:]
    pr = (pr[0] * pr[1] * pr[2]).astype(jnp.float32)
    o_ref[...] = -(s + jnp.log(pr))


_tc_call = pl.pallas_call(
    _tc_body,
    grid=(N_TC // TILE_TC,),
    in_specs=[
        pl.BlockSpec((D_OBS, TILE_TC), lambda i: (0, i)),
        pl.BlockSpec((D_OBS, D_NA), lambda i: (0, 0)),
        pl.BlockSpec((D_NA, 1), lambda i: (0, 0)),
    ],
    out_specs=pl.BlockSpec((TILE_TC,), lambda i: (i,)),
    out_shape=jax.ShapeDtypeStruct((N_TC,), jnp.float32),
)


def kernel(X, M, idxs_params, idxs_miss, coeffs, intercepts):
    order = jnp.argsort(idxs_params)
    ctab = jnp.concatenate([coeffs[order], intercepts[None, :]], axis=0)
    bits = lax.bitcast_convert_type(ctab.T.astype(jnp.bfloat16), jnp.uint16)
    word = bits.astype(jnp.uint32) | (bits.astype(jnp.uint32) << 16)
    ctab = jnp.broadcast_to(
        lax.bitcast_convert_type(word, jnp.int32).reshape(
            D_NA * (D_OBS + 1), 1), (D_NA * (D_OBS + 1), L)
    ).reshape(D_NA * (D_OBS + 1) * L)
    xt = X.T
    out_sc = _sc_call(xt, ctab)
    out_tc = _tc_call(xt, coeffs[order].astype(jnp.float32),
                      intercepts.astype(jnp.float32)[:, None])
    return jnp.concatenate([out_tc, out_sc])
